# baseline (speedup 1.0000x reference)
import functools

import jax
import jax.numpy as jnp
from jax import lax
from jax.experimental import pallas as pl
from jax.experimental.pallas import tpu as pltpu
from jax.experimental.pallas import tpu_sc as plsc

K = 8192
D = 32
N = 18432

T_BLK = 1024
SUB = 128
LANES = 128

NC, NS = 2, 16
NW = NC * NS
BPW = N // NW


def _argmin_body(z_ref, et_ref, idx_ref, esq_ref):
    i = pl.program_id(0)

    @pl.when(i == 0)
    def _():
        et = et_ref[...]
        esq_ref[...] = jnp.sum(et * et, axis=0)

    z = z_ref[...]
    col = lax.broadcasted_iota(jnp.int32, (SUB, LANES), 1)

    m2 = lax.dot_general(z + z, et_ref[...], (((1,), (0,)), ((), ())),
                         preferred_element_type=jnp.float32)

    for t0 in range(0, T_BLK, SUB):
        z_s = z[t0:t0 + SUB, :]
        zsq = jnp.sum(z_s * z_s, axis=1, keepdims=True)
        bd = None
        bi = None
        for s in range(K // LANES):
            ms = m2[t0:t0 + SUB, s * LANES:(s + 1) * LANES]
            es = esq_ref[s * LANES:(s + 1) * LANES]
            d = (zsq + es[None, :]) - ms
            if s == 0:
                bd = d
                bi = col
            else:
                upd = d < bd
                bd = jnp.where(upd, d, bd)
                bi = jnp.where(upd, col + s * LANES, bi)

        dmin = jnp.min(bd, axis=1)
        idx_ref[t0:t0 + SUB] = jnp.min(
            jnp.where(bd == dmin[:, None], bi, K), axis=1)


def _argmin_call(z_flat, emb_t):
    return pl.pallas_call(
        _argmin_body,
        grid=(N // T_BLK,),
        in_specs=[
            pl.BlockSpec((T_BLK, D), lambda i: (i, 0)),
            pl.BlockSpec((D, K), lambda i: (0, 0)),
        ],
        out_specs=pl.BlockSpec((T_BLK,), lambda i: (i,)),
        out_shape=jax.ShapeDtypeStruct((N,), jnp.int32),
        scratch_shapes=[
            pltpu.VMEM((K,), jnp.float32),
        ],
        compiler_params=pltpu.CompilerParams(
            dimension_semantics=("arbitrary",),
        ),
    )(z_flat, emb_t)


def _sc_body(table_hbm, idx_hbm, zq_hbm, hist_hbm, idx_v, rows_v, hist_v, sem):
    wid = lax.axis_index("s") * NC + lax.axis_index("c")
    base = wid * BPW
    pltpu.sync_copy(idx_hbm.at[pl.ds(base, BPW)], idx_v)
    gather = pltpu.async_copy(table_hbm.at[idx_v], rows_v, sem)

    zeros = jnp.zeros((16,), jnp.float32)

    def zero_body(i, carry):
        for u in range(8):
            hist_v[pl.ds(i * 128 + u * 16, 16)] = zeros
        return carry

    lax.fori_loop(0, K // 128, zero_body, 0, unroll=True)

    gather.wait()
    wb = pltpu.async_copy(rows_v, zq_hbm.at[pl.ds(base, BPW)], sem)

    ones = jnp.ones((16,), jnp.float32)

    def hist_body(i, carry):
        for u in range(4):
            iv = idx_v[pl.ds(i * 64 + u * 16, 16)]
            plsc.addupdate_scatter(hist_v, [iv], ones)
        return carry

    lax.fori_loop(0, BPW // 64, hist_body, 0, unroll=True)
    wb.wait()
    pltpu.sync_copy(hist_v, hist_hbm.at[wid])


@functools.cache
def _sc_gather_hist():
    return pl.kernel(
        _sc_body,
        out_type=[
            jax.ShapeDtypeStruct((N, D), jnp.float32),
            jax.ShapeDtypeStruct((NW, K), jnp.float32),
        ],
        mesh=plsc.VectorSubcoreMesh(core_axis_name="c", subcore_axis_name="s"),
        compiler_params=pltpu.CompilerParams(
            needs_layout_passes=False, use_tc_tiling_on_sc=False,
            skip_device_barrier=True),
        scratch_types=[
            pltpu.VMEM((BPW,), jnp.int32),
            pltpu.VMEM((BPW, D), jnp.float32),
            pltpu.VMEM((K,), jnp.float32),
            pltpu.SemaphoreType.DMA,
        ],
    )


def _perp_body(hist_ref, out_ref):
    counts = jnp.sum(hist_ref[...], axis=0)
    p = counts / jnp.float32(N)
    ent = jnp.sum(p * jnp.log(p + 1e-10))
    out_ref[0, 0] = jnp.exp(-ent)


def _perp_call(hist):
    return pl.pallas_call(
        _perp_body,
        out_shape=jax.ShapeDtypeStruct((1, 1), jnp.float32),
        out_specs=pl.BlockSpec(memory_space=pltpu.SMEM),
    )(hist)


def kernel(z_e, embedding):
    B, H, W, Dz = z_e.shape
    z_flat = z_e.reshape(-1, Dz)
    idx = _argmin_call(z_flat, embedding.T)
    zq_rows, hist = _sc_gather_hist()(embedding, idx)
    perplexity = _perp_call(hist)[0, 0]
    z_q = zq_rows.reshape(B, H, W, Dz)
    z_q_st = z_e + lax.stop_gradient(z_q - z_e)
    vq_loss = jnp.asarray(0.0, dtype=jnp.float32)
    commitment_loss = jnp.asarray(0.0, dtype=jnp.float32)
    indices_out = idx.reshape(B, H, W)
    return (z_q_st, indices_out, vq_loss, commitment_loss, perplexity)


# speedup vs baseline: 1.0146x; 1.0146x over previous
import functools

import jax
import jax.numpy as jnp
from jax import lax
from jax.experimental import pallas as pl
from jax.experimental.pallas import tpu as pltpu
from jax.experimental.pallas import tpu_sc as plsc

K = 8192
D = 32
N = 18432

T_BLK = 1024
SUB = 128
LANES = 128

NC, NS = 2, 16
NW = NC * NS
BPW = N // NW


def _argmin_body(z_ref, et_ref, idx_ref, esq_ref):
    i = pl.program_id(0)

    @pl.when(i == 0)
    def _():
        et = et_ref[...]
        esq_ref[...] = jnp.sum(et * et, axis=0)

    z = z_ref[...]
    col = lax.broadcasted_iota(jnp.int32, (SUB, LANES), 1)

    m2 = lax.dot_general(z + z, et_ref[...], (((1,), (0,)), ((), ())),
                         preferred_element_type=jnp.float32)

    for t0 in range(0, T_BLK, SUB):
        z_s = z[t0:t0 + SUB, :]
        zsq = jnp.sum(z_s * z_s, axis=1, keepdims=True)
        bd = None
        bi = None
        for s in range(K // LANES):
            ms = m2[t0:t0 + SUB, s * LANES:(s + 1) * LANES]
            es = esq_ref[s * LANES:(s + 1) * LANES]
            d = (zsq + es[None, :]) - ms
            if s == 0:
                bd = d
                bi = col
            else:
                upd = d < bd
                bd = jnp.where(upd, d, bd)
                bi = jnp.where(upd, col + s * LANES, bi)

        dmin = jnp.min(bd, axis=1)
        bi_f = bi.astype(jnp.float32)
        idx_ref[t0:t0 + SUB] = jnp.min(
            jnp.where(bd == dmin[:, None], bi_f, jnp.float32(K)),
            axis=1).astype(jnp.int32)


def _argmin_call(z_flat, emb_t):
    return pl.pallas_call(
        _argmin_body,
        grid=(N // T_BLK,),
        in_specs=[
            pl.BlockSpec((T_BLK, D), lambda i: (i, 0)),
            pl.BlockSpec((D, K), lambda i: (0, 0)),
        ],
        out_specs=pl.BlockSpec((T_BLK,), lambda i: (i,)),
        out_shape=jax.ShapeDtypeStruct((N,), jnp.int32),
        scratch_shapes=[
            pltpu.VMEM((K,), jnp.float32),
        ],
        compiler_params=pltpu.CompilerParams(
            dimension_semantics=("arbitrary",),
        ),
    )(z_flat, emb_t)


def _sc_body(table_hbm, idx_hbm, zq_hbm, hist_hbm, idx_v, rows_v, hist_v, sem):
    wid = lax.axis_index("s") * NC + lax.axis_index("c")
    base = wid * BPW
    pltpu.sync_copy(idx_hbm.at[pl.ds(base, BPW)], idx_v)
    gather = pltpu.async_copy(table_hbm.at[idx_v], rows_v, sem)

    zeros = jnp.zeros((16,), jnp.float32)

    def zero_body(i, carry):
        for u in range(8):
            hist_v[pl.ds(i * 128 + u * 16, 16)] = zeros
        return carry

    lax.fori_loop(0, K // 128, zero_body, 0, unroll=True)

    gather.wait()
    wb = pltpu.async_copy(rows_v, zq_hbm.at[pl.ds(base, BPW)], sem)

    ones = jnp.ones((16,), jnp.float32)

    def hist_body(i, carry):
        for u in range(4):
            iv = idx_v[pl.ds(i * 64 + u * 16, 16)]
            plsc.addupdate_scatter(hist_v, [iv], ones)
        return carry

    lax.fori_loop(0, BPW // 64, hist_body, 0, unroll=True)
    wb.wait()
    pltpu.sync_copy(hist_v, hist_hbm.at[wid])


@functools.cache
def _sc_gather_hist():
    return pl.kernel(
        _sc_body,
        out_type=[
            jax.ShapeDtypeStruct((N, D), jnp.float32),
            jax.ShapeDtypeStruct((NW, K), jnp.float32),
        ],
        mesh=plsc.VectorSubcoreMesh(core_axis_name="c", subcore_axis_name="s"),
        compiler_params=pltpu.CompilerParams(
            needs_layout_passes=False, use_tc_tiling_on_sc=False,
            skip_device_barrier=True),
        scratch_types=[
            pltpu.VMEM((BPW,), jnp.int32),
            pltpu.VMEM((BPW, D), jnp.float32),
            pltpu.VMEM((K,), jnp.float32),
            pltpu.SemaphoreType.DMA,
        ],
    )


def _perp_body(hist_ref, out_ref):
    counts = jnp.sum(hist_ref[...], axis=0)
    p = counts / jnp.float32(N)
    ent = jnp.sum(p * jnp.log(p + 1e-10))
    out_ref[0, 0] = jnp.exp(-ent)


def _perp_call(hist):
    return pl.pallas_call(
        _perp_body,
        out_shape=jax.ShapeDtypeStruct((1, 1), jnp.float32),
        out_specs=pl.BlockSpec(memory_space=pltpu.SMEM),
    )(hist)


def kernel(z_e, embedding):
    B, H, W, Dz = z_e.shape
    z_flat = z_e.reshape(-1, Dz)
    idx = _argmin_call(z_flat, embedding.T)
    zq_rows, hist = _sc_gather_hist()(embedding, idx)
    perplexity = _perp_call(hist)[0, 0]
    z_q = zq_rows.reshape(B, H, W, Dz)
    z_q_st = z_e + lax.stop_gradient(z_q - z_e)
    vq_loss = jnp.asarray(0.0, dtype=jnp.float32)
    commitment_loss = jnp.asarray(0.0, dtype=jnp.float32)
    indices_out = idx.reshape(B, H, W)
    return (z_q_st, indices_out, vq_loss, commitment_loss, perplexity)
